# Initial kernel scaffold; baseline (speedup 1.0000x reference)
import jax
import jax.numpy as jnp
from jax import lax
from jax.experimental import pallas as pl
from jax.experimental.pallas import tpu as pltpu
from jax.experimental.pallas import tpu_sc as plsc

N = 10000
E = 320000
G = 64
D = 128
D_OUT = 64
NUM_LAYERS = 5

NC = 2
NS = 16
NW = NC * NS

NP = 10240
EPT = 10240
EP = NW * EPT
CHUNK = 128
STEPS = EPT // CHUNK

_MESH = plsc.VectorSubcoreMesh(core_axis_name="c", subcore_axis_name="s")


def _deg_body(dst_hbm, ones_hbm, zeros_hbm, out_hbm, idx_d, ones_v, acc, sem):
    c = lax.axis_index("c")
    s = lax.axis_index("s")
    wid = c * NS + s
    pltpu.sync_copy(zeros_hbm, acc.at[pl.ds(s * 640, 640)])
    pltpu.sync_copy(ones_hbm, ones_v)
    plsc.subcore_barrier()

    eb = wid * EPT

    def step(k, carry):
        off = eb + k * CHUNK
        pltpu.sync_copy(dst_hbm.at[pl.ds(off, CHUNK)], idx_d)
        pltpu.sync_copy(ones_v, acc.at[idx_d], add=True)
        return carry

    lax.fori_loop(0, STEPS, step, 0)
    plsc.subcore_barrier()
    pltpu.sync_copy(acc.at[pl.ds(s * 640, 640)],
                    out_hbm.at[c, pl.ds(s * 640, 640)])


_deg_kernel = pl.kernel(
    _deg_body,
    out_type=jax.ShapeDtypeStruct((NC, NP, 16), jnp.float32),
    mesh=_MESH,
    scratch_types=[
        pltpu.VMEM((CHUNK,), jnp.int32),
        pltpu.VMEM((CHUNK, 16), jnp.float32),
        pltpu.VMEM_SHARED((NP, 16), jnp.float32),
        pltpu.SemaphoreType.DMA,
    ],
)


def _agg_body(src_hbm, dst_hbm, table_hbm, zeros_hbm, out_hbm,
              idx_s, idx_d, rows, acc, sem):
    c = lax.axis_index("c")
    s = lax.axis_index("s")
    wid = c * NS + s
    pltpu.sync_copy(zeros_hbm, acc.at[pl.ds(s * 640, 640)])
    plsc.subcore_barrier()

    eb = wid * EPT

    def step(k, carry):
        off = eb + k * CHUNK
        pltpu.sync_copy(src_hbm.at[pl.ds(off, CHUNK)], idx_s)
        pltpu.sync_copy(dst_hbm.at[pl.ds(off, CHUNK)], idx_d)
        pltpu.async_copy(table_hbm.at[idx_s], rows, sem).wait()
        pltpu.sync_copy(rows, acc.at[idx_d], add=True)
        return carry

    lax.fori_loop(0, STEPS, step, 0)
    plsc.subcore_barrier()
    pltpu.sync_copy(acc.at[pl.ds(s * 640, 640)],
                    out_hbm.at[c, pl.ds(s * 640, 640)])


_agg_kernel = pl.kernel(
    _agg_body,
    out_type=jax.ShapeDtypeStruct((NC, NP, D), jnp.float32),
    mesh=_MESH,
    scratch_types=[
        pltpu.VMEM((CHUNK,), jnp.int32),
        pltpu.VMEM((CHUNK,), jnp.int32),
        pltpu.VMEM((CHUNK, D), jnp.float32),
        pltpu.VMEM_SHARED((NP, D), jnp.float32),
        pltpu.SemaphoreType.DMA,
    ],
)


def _first_body(x_ref, w_ref, d0_ref, d1_ref, hpre_ref, hs_ref, dinv_ref):
    deg = d0_ref[0:N, 0:1] + d1_ref[0:N, 0:1] + 1.0
    dinv = lax.rsqrt(deg)
    dinv_ref[...] = dinv
    h = jnp.dot(x_ref[...], w_ref[...], preferred_element_type=jnp.float32)
    hpre_ref[...] = h
    hs_ref[0:N, :] = h * dinv
    hs_ref[N:NP, :] = jnp.zeros((NP - N, D), jnp.float32)


_first_kernel = pl.pallas_call(
    _first_body,
    out_shape=(
        jax.ShapeDtypeStruct((N, D), jnp.float32),
        jax.ShapeDtypeStruct((NP, D), jnp.float32),
        jax.ShapeDtypeStruct((N, 1), jnp.float32),
    ),
)


def _mid_body(p0_ref, p1_ref, hpre_ref, dinv_ref, b_ref, g_ref, be_ref,
              rm_ref, rv_ref, w_ref, hpre_o_ref, hs_o_ref):
    dinv = dinv_ref[...]
    agg = (p0_ref[0:N, :] + p1_ref[0:N, :]) * dinv \
        + hpre_ref[...] * (dinv * dinv) + b_ref[...]
    o = (agg - rm_ref[...]) * lax.rsqrt(rv_ref[...] + 1e-5) * g_ref[...] \
        + be_ref[...]
    o = jnp.maximum(o, 0.0)
    h = jnp.dot(o, w_ref[...], preferred_element_type=jnp.float32)
    hpre_o_ref[...] = h
    hs_o_ref[0:N, :] = h * dinv
    hs_o_ref[N:NP, :] = jnp.zeros((NP - N, D), jnp.float32)


_mid_kernel = pl.pallas_call(
    _mid_body,
    out_shape=(
        jax.ShapeDtypeStruct((N, D), jnp.float32),
        jax.ShapeDtypeStruct((NP, D), jnp.float32),
    ),
)


def _final_body(p0_ref, p1_ref, hpre_ref, dinv_ref, b_ref, batch_ref,
                wout_ref, bout_ref, out_ref):
    dinv = dinv_ref[...]
    o = (p0_ref[0:N, :] + p1_ref[0:N, :]) * dinv \
        + hpre_ref[...] * (dinv * dinv) + b_ref[...]
    gids = lax.broadcasted_iota(jnp.int32, (1, G), 1)
    onehot = jnp.where(batch_ref[...] == gids, 1.0, 0.0)
    sums = lax.dot_general(onehot, o, (((0,), (0,)), ((), ())),
                           preferred_element_type=jnp.float32)
    counts = lax.dot_general(onehot, jnp.ones((N, 1), jnp.float32),
                             (((0,), (0,)), ((), ())),
                             preferred_element_type=jnp.float32)
    pooled = sums / jnp.maximum(counts, 1.0)
    logits = jnp.dot(pooled, wout_ref[...],
                     preferred_element_type=jnp.float32) + bout_ref[...]
    m = jnp.max(logits, axis=1, keepdims=True)
    z = logits - m
    lse = jnp.log(jnp.sum(jnp.exp(z), axis=1, keepdims=True))
    out_ref[...] = z - lse


_final_kernel = pl.pallas_call(
    _final_body,
    out_shape=jax.ShapeDtypeStruct((G, D_OUT), jnp.float32),
)


def kernel(x, edge_index, batch, Ws, bs, gammas, betas, rms, rvs, Wout, bout):
    pad = jnp.full((EP - E,), N, jnp.int32)
    src_p = jnp.concatenate([edge_index[0].astype(jnp.int32), pad])
    dst_p = jnp.concatenate([edge_index[1].astype(jnp.int32), pad])

    ones16 = jnp.ones((CHUNK, 16), jnp.float32)
    zeros16 = jnp.zeros((640, 16), jnp.float32)
    zerosD = jnp.zeros((640, D), jnp.float32)

    degp = _deg_kernel(dst_p, ones16, zeros16)

    hpre, hs, dinv = _first_kernel(x.astype(jnp.float32), Ws[0],
                                   degp[0], degp[1])

    for i in range(1, NUM_LAYERS):
        p = _agg_kernel(src_p, dst_p, hs, zerosD)
        j = i - 1
        hpre, hs = _mid_kernel(
            p[0], p[1], hpre, dinv,
            bs[j].reshape(1, D), gammas[j].reshape(1, D),
            betas[j].reshape(1, D), rms[j].reshape(1, D),
            rvs[j].reshape(1, D), Ws[i])

    p = _agg_kernel(src_p, dst_p, hs, zerosD)
    return _final_kernel(p[0], p[1], hpre, dinv,
                         bs[NUM_LAYERS - 1].reshape(1, D),
                         batch.reshape(N, 1).astype(jnp.int32),
                         Wout, bout.reshape(1, D_OUT))


# speedup vs baseline: 4.9373x; 4.9373x over previous
import jax
import jax.numpy as jnp
from jax import lax
from jax.experimental import pallas as pl
from jax.experimental.pallas import tpu as pltpu
from jax.experimental.pallas import tpu_sc as plsc

N = 10000
E = 320000
G = 64
D = 128
D_OUT = 64
NUM_LAYERS = 5

NC = 2
NS = 16
NW = NC * NS

NP = 10240
EPT = 10240
EP = NW * EPT
CHUNK = 128
STEPS = EPT // CHUNK

import functools


@functools.lru_cache(maxsize=None)
def _mesh():
    return plsc.VectorSubcoreMesh(core_axis_name="c", subcore_axis_name="s",
                                  num_cores=NC, num_subcores=NS)


def _deg_body(dst_hbm, ones_hbm, zeros_hbm, out_hbm, idx_d, ones_v, acc, sem):
    c = lax.axis_index("c")
    s = lax.axis_index("s")
    wid = c * NS + s
    pltpu.sync_copy(zeros_hbm, acc.at[pl.ds(s * 640, 640)])
    pltpu.sync_copy(ones_hbm, ones_v)
    plsc.subcore_barrier()

    eb = wid * EPT

    def step(k, carry):
        off = eb + k * CHUNK
        pltpu.sync_copy(dst_hbm.at[pl.ds(off, CHUNK)], idx_d)
        pltpu.sync_copy(ones_v, acc.at[idx_d], add=True)
        return carry

    lax.fori_loop(0, STEPS, step, 0)
    plsc.subcore_barrier()
    pltpu.sync_copy(acc.at[pl.ds(s * 640, 640)],
                    out_hbm.at[c, pl.ds(s * 640, 640)])


@functools.lru_cache(maxsize=None)
def _deg_kernel():
    return pl.kernel(
        _deg_body,
        out_type=jax.ShapeDtypeStruct((NC, NP, D), jnp.float32),
        mesh=_mesh(),
        scratch_types=[
            pltpu.VMEM((CHUNK,), jnp.int32),
            pltpu.VMEM((CHUNK, D), jnp.float32),
            pltpu.VMEM_SHARED((NP, D), jnp.float32),
            pltpu.SemaphoreType.DMA,
        ],
    )


def _agg_body(src_hbm, dst_hbm, table_hbm, zeros_hbm, out_hbm,
              idx_s, idx_d, rows, acc, sem):
    c = lax.axis_index("c")
    s = lax.axis_index("s")
    wid = c * NS + s
    pltpu.sync_copy(zeros_hbm, acc.at[pl.ds(s * 640, 640)])
    plsc.subcore_barrier()

    eb = wid * EPT

    def step(k, carry):
        off = eb + k * CHUNK
        pltpu.sync_copy(src_hbm.at[pl.ds(off, CHUNK)], idx_s)
        pltpu.sync_copy(dst_hbm.at[pl.ds(off, CHUNK)], idx_d)
        pltpu.async_copy(table_hbm.at[idx_s], rows, sem).wait()
        pltpu.sync_copy(rows, acc.at[idx_d], add=True)
        return carry

    lax.fori_loop(0, STEPS, step, 0)
    plsc.subcore_barrier()
    pltpu.sync_copy(acc.at[pl.ds(s * 640, 640)],
                    out_hbm.at[c, pl.ds(s * 640, 640)])


@functools.lru_cache(maxsize=None)
def _agg_kernel():
    return pl.kernel(
        _agg_body,
        out_type=jax.ShapeDtypeStruct((NC, NP, D), jnp.float32),
        mesh=_mesh(),
        scratch_types=[
            pltpu.VMEM((CHUNK,), jnp.int32),
            pltpu.VMEM((CHUNK,), jnp.int32),
            pltpu.VMEM((CHUNK, D), jnp.float32),
            pltpu.VMEM_SHARED((NP, D), jnp.float32),
            pltpu.SemaphoreType.DMA,
        ],
    )


def _first_body(x_ref, w_ref, d0_ref, d1_ref, hpre_ref, hs_ref, dinv_ref):
    deg = d0_ref[0:N, 0:1] + d1_ref[0:N, 0:1] + 1.0
    dinv = lax.rsqrt(deg)
    dinv_ref[...] = dinv
    h = jnp.dot(x_ref[...], w_ref[...], preferred_element_type=jnp.float32)
    hpre_ref[...] = h
    hs_ref[0:N, :] = h * dinv
    hs_ref[N:NP, :] = jnp.zeros((NP - N, D), jnp.float32)


_first_kernel = pl.pallas_call(
    _first_body,
    out_shape=(
        jax.ShapeDtypeStruct((N, D), jnp.float32),
        jax.ShapeDtypeStruct((NP, D), jnp.float32),
        jax.ShapeDtypeStruct((N, 1), jnp.float32),
    ),
)


def _mid_body(p0_ref, p1_ref, hpre_ref, dinv_ref, b_ref, g_ref, be_ref,
              rm_ref, rv_ref, w_ref, hpre_o_ref, hs_o_ref):
    dinv = dinv_ref[...]
    agg = (p0_ref[0:N, :] + p1_ref[0:N, :]) * dinv \
        + hpre_ref[...] * (dinv * dinv) + b_ref[...]
    o = (agg - rm_ref[...]) * lax.rsqrt(rv_ref[...] + 1e-5) * g_ref[...] \
        + be_ref[...]
    o = jnp.maximum(o, 0.0)
    h = jnp.dot(o, w_ref[...], preferred_element_type=jnp.float32)
    hpre_o_ref[...] = h
    hs_o_ref[0:N, :] = h * dinv
    hs_o_ref[N:NP, :] = jnp.zeros((NP - N, D), jnp.float32)


_mid_kernel = pl.pallas_call(
    _mid_body,
    out_shape=(
        jax.ShapeDtypeStruct((N, D), jnp.float32),
        jax.ShapeDtypeStruct((NP, D), jnp.float32),
    ),
)


def _final_body(p0_ref, p1_ref, hpre_ref, dinv_ref, b_ref, batch_ref,
                wout_ref, bout_ref, out_ref):
    dinv = dinv_ref[...]
    o = (p0_ref[0:N, :] + p1_ref[0:N, :]) * dinv \
        + hpre_ref[...] * (dinv * dinv) + b_ref[...]
    gids = lax.broadcasted_iota(jnp.int32, (1, G), 1)
    onehot = jnp.where(batch_ref[...] == gids, 1.0, 0.0)
    sums = lax.dot_general(onehot, o, (((0,), (0,)), ((), ())),
                           preferred_element_type=jnp.float32)
    counts = lax.dot_general(onehot, jnp.ones((N, 1), jnp.float32),
                             (((0,), (0,)), ((), ())),
                             preferred_element_type=jnp.float32)
    pooled = sums / jnp.maximum(counts, 1.0)
    logits = jnp.dot(pooled, wout_ref[...],
                     preferred_element_type=jnp.float32) + bout_ref[...]
    m = jnp.max(logits, axis=1, keepdims=True)
    z = logits - m
    lse = jnp.log(jnp.sum(jnp.exp(z), axis=1, keepdims=True))
    out_ref[...] = z - lse


_final_kernel = pl.pallas_call(
    _final_body,
    out_shape=jax.ShapeDtypeStruct((G, D_OUT), jnp.float32),
)


def kernel(x, edge_index, batch, Ws, bs, gammas, betas, rms, rvs, Wout, bout):
    pad = jnp.full((EP - E,), N, jnp.int32)
    src_p = jnp.concatenate([edge_index[0].astype(jnp.int32), pad])
    dst_p = jnp.concatenate([edge_index[1].astype(jnp.int32), pad])

    onesD = jnp.ones((CHUNK, D), jnp.float32)
    zerosD = jnp.zeros((640, D), jnp.float32)

    degp = _deg_kernel()(dst_p, onesD, zerosD)

    hpre, hs, dinv = _first_kernel(x.astype(jnp.float32), Ws[0],
                                   degp[0], degp[1])

    for i in range(1, NUM_LAYERS):
        p = _agg_kernel()(src_p, dst_p, hs, zerosD)
        j = i - 1
        hpre, hs = _mid_kernel(
            p[0], p[1], hpre, dinv,
            bs[j].reshape(1, D), gammas[j].reshape(1, D),
            betas[j].reshape(1, D), rms[j].reshape(1, D),
            rvs[j].reshape(1, D), Ws[i])

    p = _agg_kernel()(src_p, dst_p, hs, zerosD)
    return _final_kernel(p[0], p[1], hpre, dinv,
                         bs[NUM_LAYERS - 1].reshape(1, D),
                         batch.reshape(N, 1).astype(jnp.int32),
                         Wout, bout.reshape(1, D_OUT))


# speedup vs baseline: 5.7113x; 1.1568x over previous
import jax
import jax.numpy as jnp
from jax import lax
from jax.experimental import pallas as pl
from jax.experimental.pallas import tpu as pltpu
from jax.experimental.pallas import tpu_sc as plsc

N = 10000
E = 320000
G = 64
D = 128
D_OUT = 64
NUM_LAYERS = 5

NC = 2
NS = 16
NW = NC * NS

NP = 10240
EPT = 10240
EP = NW * EPT
CHUNK = 128
STEPS = EPT // CHUNK

import functools


@functools.lru_cache(maxsize=None)
def _mesh():
    return plsc.VectorSubcoreMesh(core_axis_name="c", subcore_axis_name="s",
                                  num_cores=NC, num_subcores=NS)


NBUF = 2
HSTEPS = 40


def _deg_body(dst_hbm, ones_hbm, zeros_hbm, out_hbm, idx_all, ones_v, acc,
              sem):
    c = lax.axis_index("c")
    s = lax.axis_index("s")
    wid = c * NS + s
    pltpu.sync_copy(zeros_hbm, acc.at[pl.ds(s * 640, 640)])
    pltpu.sync_copy(ones_hbm, ones_v)
    pltpu.sync_copy(dst_hbm.at[wid], idx_all)
    plsc.subcore_barrier()

    def block(t, carry):
        descs = []
        for j in range(NBUF):
            k = t * NBUF + j
            descs.append(pltpu.async_copy(
                ones_v, acc.at[idx_all.at[k]], sem.at[j], add=True))
        for d in descs:
            d.wait()
        return carry

    lax.fori_loop(0, STEPS // NBUF, block, 0)
    plsc.subcore_barrier()
    pltpu.sync_copy(acc.at[pl.ds(s * 640, 640)],
                    out_hbm.at[c, pl.ds(s * 640, 640)])


@functools.lru_cache(maxsize=None)
def _deg_kernel():
    return pl.kernel(
        _deg_body,
        out_type=jax.ShapeDtypeStruct((NC, NP, D), jnp.float32),
        mesh=_mesh(),
        scratch_types=[
            pltpu.VMEM((STEPS, CHUNK), jnp.int32),
            pltpu.VMEM((CHUNK, D), jnp.float32),
            pltpu.VMEM_SHARED((NP, D), jnp.float32),
            pltpu.SemaphoreType.DMA((NBUF,)),
        ],
    )


def _agg_body(edges_hbm, table_hbm, zeros_hbm, out_hbm,
              idx_all, rows, acc, gsem, ssem):
    c = lax.axis_index("c")
    s = lax.axis_index("s")
    wid = c * NS + s
    pltpu.sync_copy(zeros_hbm, acc.at[pl.ds(s * 640, 640)])
    plsc.subcore_barrier()

    def phase(half, carry0):
        pltpu.sync_copy(edges_hbm.at[wid, :, pl.ds(half * HSTEPS, HSTEPS)],
                        idx_all)

        def block(t, carry):
            gds = []
            for j in range(NBUF):
                k = t * NBUF + j
                gds.append(pltpu.async_copy(
                    table_hbm.at[idx_all.at[0, k]], rows.at[j], gsem.at[j]))
            sds = []
            for j in range(NBUF):
                k = t * NBUF + j
                gds[j].wait()
                sds.append(pltpu.async_copy(
                    rows.at[j], acc.at[idx_all.at[1, k]], ssem.at[j],
                    add=True))
            for d in sds:
                d.wait()
            return carry

        lax.fori_loop(0, HSTEPS // NBUF, block, 0)
        return carry0

    lax.fori_loop(0, STEPS // HSTEPS, phase, 0)
    plsc.subcore_barrier()
    pltpu.sync_copy(acc.at[pl.ds(s * 640, 640)],
                    out_hbm.at[c, pl.ds(s * 640, 640)])


@functools.lru_cache(maxsize=None)
def _agg_kernel():
    return pl.kernel(
        _agg_body,
        out_type=jax.ShapeDtypeStruct((NC, NP, D), jnp.float32),
        mesh=_mesh(),
        scratch_types=[
            pltpu.VMEM((2, HSTEPS, CHUNK), jnp.int32),
            pltpu.VMEM((NBUF, CHUNK, D), jnp.float32),
            pltpu.VMEM_SHARED((NP, D), jnp.float32),
            pltpu.SemaphoreType.DMA((NBUF,)),
            pltpu.SemaphoreType.DMA((NBUF,)),
        ],
    )


def _first_body(x_ref, w_ref, d0_ref, d1_ref, hpre_ref, hs_ref, dinv_ref):
    deg = d0_ref[0:N, 0:1] + d1_ref[0:N, 0:1] + 1.0
    dinv = lax.rsqrt(deg)
    dinv_ref[...] = dinv
    h = jnp.dot(x_ref[...], w_ref[...], preferred_element_type=jnp.float32)
    hpre_ref[...] = h
    hs_ref[0:N, :] = h * dinv
    hs_ref[N:NP, :] = jnp.zeros((NP - N, D), jnp.float32)


_first_kernel = pl.pallas_call(
    _first_body,
    out_shape=(
        jax.ShapeDtypeStruct((N, D), jnp.float32),
        jax.ShapeDtypeStruct((NP, D), jnp.float32),
        jax.ShapeDtypeStruct((N, 1), jnp.float32),
    ),
)


def _mid_body(p0_ref, p1_ref, hpre_ref, dinv_ref, b_ref, g_ref, be_ref,
              rm_ref, rv_ref, w_ref, hpre_o_ref, hs_o_ref):
    dinv = dinv_ref[...]
    agg = (p0_ref[0:N, :] + p1_ref[0:N, :]) * dinv \
        + hpre_ref[...] * (dinv * dinv) + b_ref[...]
    o = (agg - rm_ref[...]) * lax.rsqrt(rv_ref[...] + 1e-5) * g_ref[...] \
        + be_ref[...]
    o = jnp.maximum(o, 0.0)
    h = jnp.dot(o, w_ref[...], preferred_element_type=jnp.float32)
    hpre_o_ref[...] = h
    hs_o_ref[0:N, :] = h * dinv
    hs_o_ref[N:NP, :] = jnp.zeros((NP - N, D), jnp.float32)


_mid_kernel = pl.pallas_call(
    _mid_body,
    out_shape=(
        jax.ShapeDtypeStruct((N, D), jnp.float32),
        jax.ShapeDtypeStruct((NP, D), jnp.float32),
    ),
)


def _final_body(p0_ref, p1_ref, hpre_ref, dinv_ref, b_ref, batch_ref,
                wout_ref, bout_ref, out_ref):
    dinv = dinv_ref[...]
    o = (p0_ref[0:N, :] + p1_ref[0:N, :]) * dinv \
        + hpre_ref[...] * (dinv * dinv) + b_ref[...]
    gids = lax.broadcasted_iota(jnp.int32, (1, G), 1)
    onehot = jnp.where(batch_ref[...] == gids, 1.0, 0.0)
    sums = lax.dot_general(onehot, o, (((0,), (0,)), ((), ())),
                           preferred_element_type=jnp.float32)
    counts = lax.dot_general(onehot, jnp.ones((N, 1), jnp.float32),
                             (((0,), (0,)), ((), ())),
                             preferred_element_type=jnp.float32)
    pooled = sums / jnp.maximum(counts, 1.0)
    logits = jnp.dot(pooled, wout_ref[...],
                     preferred_element_type=jnp.float32) + bout_ref[...]
    m = jnp.max(logits, axis=1, keepdims=True)
    z = logits - m
    lse = jnp.log(jnp.sum(jnp.exp(z), axis=1, keepdims=True))
    out_ref[...] = z - lse


_final_kernel = pl.pallas_call(
    _final_body,
    out_shape=jax.ShapeDtypeStruct((G, D_OUT), jnp.float32),
)


def kernel(x, edge_index, batch, Ws, bs, gammas, betas, rms, rvs, Wout, bout):
    pad = jnp.full((EP - E,), N, jnp.int32)
    src_p = jnp.concatenate([edge_index[0].astype(jnp.int32), pad])
    dst_p = jnp.concatenate([edge_index[1].astype(jnp.int32), pad])
    src_r = src_p.reshape(NW, STEPS, CHUNK)
    dst_r = dst_p.reshape(NW, STEPS, CHUNK)
    edges3 = jnp.stack([src_r, dst_r], axis=1)

    onesD = jnp.ones((CHUNK, D), jnp.float32)
    zerosD = jnp.zeros((640, D), jnp.float32)

    degp = _deg_kernel()(dst_r, onesD, zerosD)

    hpre, hs, dinv = _first_kernel(x.astype(jnp.float32), Ws[0],
                                   degp[0], degp[1])

    for i in range(1, NUM_LAYERS):
        p = _agg_kernel()(edges3, hs, zerosD)
        j = i - 1
        hpre, hs = _mid_kernel(
            p[0], p[1], hpre, dinv,
            bs[j].reshape(1, D), gammas[j].reshape(1, D),
            betas[j].reshape(1, D), rms[j].reshape(1, D),
            rvs[j].reshape(1, D), Ws[i])

    p = _agg_kernel()(edges3, hs, zerosD)
    return _final_kernel(p[0], p[1], hpre, dinv,
                         bs[NUM_LAYERS - 1].reshape(1, D),
                         batch.reshape(N, 1).astype(jnp.int32),
                         Wout, bout.reshape(1, D_OUT))


# speedup vs baseline: 6.2035x; 1.0862x over previous
import jax
import jax.numpy as jnp
from jax import lax
from jax.experimental import pallas as pl
from jax.experimental.pallas import tpu as pltpu
from jax.experimental.pallas import tpu_sc as plsc

N = 10000
E = 320000
G = 64
D = 128
D_OUT = 64
NUM_LAYERS = 5

NC = 2
NS = 16
NW = NC * NS

NP = 10240
EPT = 10240
EP = NW * EPT
CHUNK = 128
STEPS = EPT // CHUNK

import functools


@functools.lru_cache(maxsize=None)
def _mesh():
    return plsc.VectorSubcoreMesh(core_axis_name="c", subcore_axis_name="s",
                                  num_cores=NC, num_subcores=NS)


NBUF = 2
HSTEPS = 40


def _deg_body(dst_hbm, out_hbm, idx_all, dacc, sem):
    c = lax.axis_index("c")
    s = lax.axis_index("s")
    wid = c * NS + s
    pltpu.sync_copy(dst_hbm.at[wid], idx_all)

    z = jnp.zeros((16,), jnp.float32)

    def zstep(i, carry):
        dacc[pl.ds(i * 16, 16)] = z
        return carry

    lax.fori_loop(0, NP // 16, zstep, 0)

    ones = jnp.ones((16,), jnp.float32)

    def step(k, carry):
        for j in range(CHUNK // 16):
            idx = idx_all[k, pl.ds(j * 16, 16)]
            plsc.addupdate_scatter(dacc, [idx], ones)
        return carry

    lax.fori_loop(0, STEPS, step, 0)
    pltpu.sync_copy(dacc, out_hbm.at[wid])


@functools.lru_cache(maxsize=None)
def _deg_kernel():
    return pl.kernel(
        _deg_body,
        out_type=jax.ShapeDtypeStruct((NW, NP), jnp.float32),
        mesh=_mesh(),
        compiler_params=pltpu.CompilerParams(needs_layout_passes=False),
        scratch_types=[
            pltpu.VMEM((STEPS, CHUNK), jnp.int32),
            pltpu.VMEM((NP,), jnp.float32),
            pltpu.SemaphoreType.DMA,
        ],
    )


def _agg_body(edges_hbm, table_hbm, zeros_hbm, out_hbm,
              idx_all, rows, acc, gsem, ssem):
    c = lax.axis_index("c")
    s = lax.axis_index("s")
    wid = c * NS + s
    pltpu.sync_copy(zeros_hbm, acc.at[pl.ds(s * 640, 640)])
    plsc.subcore_barrier()

    def phase(half, carry0):
        pltpu.sync_copy(edges_hbm.at[wid, :, pl.ds(half * HSTEPS, HSTEPS)],
                        idx_all)

        def block(t, carry):
            gds = []
            for j in range(NBUF):
                k = t * NBUF + j
                gds.append(pltpu.async_copy(
                    table_hbm.at[idx_all.at[0, k]], rows.at[j], gsem.at[j]))
            sds = []
            for j in range(NBUF):
                k = t * NBUF + j
                gds[j].wait()
                sds.append(pltpu.async_copy(
                    rows.at[j], acc.at[idx_all.at[1, k]], ssem.at[j],
                    add=True))
            for d in sds:
                d.wait()
            return carry

        lax.fori_loop(0, HSTEPS // NBUF, block, 0)
        return carry0

    lax.fori_loop(0, STEPS // HSTEPS, phase, 0)
    plsc.subcore_barrier()
    pltpu.sync_copy(acc.at[pl.ds(s * 640, 640)],
                    out_hbm.at[c, pl.ds(s * 640, 640)])


@functools.lru_cache(maxsize=None)
def _agg_kernel():
    return pl.kernel(
        _agg_body,
        out_type=jax.ShapeDtypeStruct((NC, NP, D), jnp.float32),
        mesh=_mesh(),
        scratch_types=[
            pltpu.VMEM((2, HSTEPS, CHUNK), jnp.int32),
            pltpu.VMEM((NBUF, CHUNK, D), jnp.float32),
            pltpu.VMEM_SHARED((NP, D), jnp.float32),
            pltpu.SemaphoreType.DMA((NBUF,)),
            pltpu.SemaphoreType.DMA((NBUF,)),
        ],
    )


def _first_body(x_ref, w_ref, dp_ref, hpre_ref, hs_ref, dinv_ref):
    dsum = lax.dot_general(dp_ref[...], jnp.ones((NW, 1), jnp.float32),
                           (((0,), (0,)), ((), ())),
                           preferred_element_type=jnp.float32)
    deg = dsum[0:N, :] + 1.0
    dinv = lax.rsqrt(deg)
    dinv_ref[...] = dinv
    h = jnp.dot(x_ref[...], w_ref[...], preferred_element_type=jnp.float32)
    hpre_ref[...] = h
    hs_ref[0:N, :] = h * dinv
    hs_ref[N:NP, :] = jnp.zeros((NP - N, D), jnp.float32)


_first_kernel = pl.pallas_call(
    _first_body,
    out_shape=(
        jax.ShapeDtypeStruct((N, D), jnp.float32),
        jax.ShapeDtypeStruct((NP, D), jnp.float32),
        jax.ShapeDtypeStruct((N, 1), jnp.float32),
    ),
)


def _mid_body(p0_ref, p1_ref, hpre_ref, dinv_ref, b_ref, g_ref, be_ref,
              rm_ref, rv_ref, w_ref, hpre_o_ref, hs_o_ref):
    dinv = dinv_ref[...]
    agg = (p0_ref[0:N, :] + p1_ref[0:N, :]) * dinv \
        + hpre_ref[...] * (dinv * dinv) + b_ref[...]
    o = (agg - rm_ref[...]) * lax.rsqrt(rv_ref[...] + 1e-5) * g_ref[...] \
        + be_ref[...]
    o = jnp.maximum(o, 0.0)
    h = jnp.dot(o, w_ref[...], preferred_element_type=jnp.float32)
    hpre_o_ref[...] = h
    hs_o_ref[0:N, :] = h * dinv
    hs_o_ref[N:NP, :] = jnp.zeros((NP - N, D), jnp.float32)


_mid_kernel = pl.pallas_call(
    _mid_body,
    out_shape=(
        jax.ShapeDtypeStruct((N, D), jnp.float32),
        jax.ShapeDtypeStruct((NP, D), jnp.float32),
    ),
)


def _final_body(p0_ref, p1_ref, hpre_ref, dinv_ref, b_ref, batch_ref,
                wout_ref, bout_ref, out_ref):
    dinv = dinv_ref[...]
    o = (p0_ref[0:N, :] + p1_ref[0:N, :]) * dinv \
        + hpre_ref[...] * (dinv * dinv) + b_ref[...]
    gids = lax.broadcasted_iota(jnp.int32, (1, G), 1)
    onehot = jnp.where(batch_ref[...] == gids, 1.0, 0.0)
    sums = lax.dot_general(onehot, o, (((0,), (0,)), ((), ())),
                           preferred_element_type=jnp.float32)
    counts = lax.dot_general(onehot, jnp.ones((N, 1), jnp.float32),
                             (((0,), (0,)), ((), ())),
                             preferred_element_type=jnp.float32)
    pooled = sums / jnp.maximum(counts, 1.0)
    logits = jnp.dot(pooled, wout_ref[...],
                     preferred_element_type=jnp.float32) + bout_ref[...]
    m = jnp.max(logits, axis=1, keepdims=True)
    z = logits - m
    lse = jnp.log(jnp.sum(jnp.exp(z), axis=1, keepdims=True))
    out_ref[...] = z - lse


_final_kernel = pl.pallas_call(
    _final_body,
    out_shape=jax.ShapeDtypeStruct((G, D_OUT), jnp.float32),
)


def kernel(x, edge_index, batch, Ws, bs, gammas, betas, rms, rvs, Wout, bout):
    pad = jnp.full((EP - E,), N, jnp.int32)
    src_p = jnp.concatenate([edge_index[0].astype(jnp.int32), pad])
    dst_p = jnp.concatenate([edge_index[1].astype(jnp.int32), pad])
    src_r = src_p.reshape(NW, STEPS, CHUNK)
    dst_r = dst_p.reshape(NW, STEPS, CHUNK)
    edges3 = jnp.stack([src_r, dst_r], axis=1)

    zerosD = jnp.zeros((640, D), jnp.float32)

    degp = _deg_kernel()(dst_r)

    hpre, hs, dinv = _first_kernel(x.astype(jnp.float32), Ws[0], degp)

    for i in range(1, NUM_LAYERS):
        p = _agg_kernel()(edges3, hs, zerosD)
        j = i - 1
        hpre, hs = _mid_kernel(
            p[0], p[1], hpre, dinv,
            bs[j].reshape(1, D), gammas[j].reshape(1, D),
            betas[j].reshape(1, D), rms[j].reshape(1, D),
            rvs[j].reshape(1, D), Ws[i])

    p = _agg_kernel()(edges3, hs, zerosD)
    return _final_kernel(p[0], p[1], hpre, dinv,
                         bs[NUM_LAYERS - 1].reshape(1, D),
                         batch.reshape(N, 1).astype(jnp.int32),
                         Wout, bout.reshape(1, D_OUT))


# speedup vs baseline: 9.8429x; 1.5867x over previous
import jax
import jax.numpy as jnp
from jax import lax
from jax.experimental import pallas as pl
from jax.experimental.pallas import tpu as pltpu
from jax.experimental.pallas import tpu_sc as plsc

N = 10000
E = 320000
G = 64
D = 128
D_OUT = 64
NUM_LAYERS = 5

NC = 2
NS = 16
NW = NC * NS

NP = 10240
EPT = 10240
EP = NW * EPT
CHUNK = 128
STEPS = EPT // CHUNK

import functools


@functools.lru_cache(maxsize=None)
def _mesh():
    return plsc.VectorSubcoreMesh(core_axis_name="c", subcore_axis_name="s",
                                  num_cores=NC, num_subcores=NS)


NBUF = 2
HALFR = 5120
TROWS = 328
ACCR = NS * TROWS
DUMMY = HALFR
SEG = EP // NS
SSTEPS = SEG // CHUNK
PH = 40
CAP = SEG + NBUF * CHUNK
PADDST = 2 * NP


def _deg_body(dst_hbm, out_hbm, idx_all, dacc, sem):
    c = lax.axis_index("c")
    s = lax.axis_index("s")
    wid = c * NS + s
    pltpu.sync_copy(dst_hbm.at[wid], idx_all)

    z = jnp.zeros((16,), jnp.float32)

    def zstep(i, carry):
        dacc[pl.ds(i * 16, 16)] = z
        return carry

    lax.fori_loop(0, NP // 16, zstep, 0)

    ones = jnp.ones((16,), jnp.float32)

    def step(k, carry):
        for j in range(CHUNK // 16):
            idx = idx_all[k, pl.ds(j * 16, 16)]
            plsc.addupdate_scatter(dacc, [idx], ones)
        return carry

    lax.fori_loop(0, STEPS, step, 0)
    pltpu.sync_copy(dacc, out_hbm.at[wid])


@functools.lru_cache(maxsize=None)
def _deg_kernel():
    return pl.kernel(
        _deg_body,
        out_type=jax.ShapeDtypeStruct((NW, NP), jnp.float32),
        mesh=_mesh(),
        compiler_params=pltpu.CompilerParams(needs_layout_passes=False),
        scratch_types=[
            pltpu.VMEM((STEPS, CHUNK), jnp.int32),
            pltpu.VMEM((NP,), jnp.float32),
            pltpu.SemaphoreType.DMA,
        ],
    )


def _agg_body(edges_hbm, table_hbm, zeros_hbm, out_hbm,
              idx_stage, kept_src, kept_dst, rows, sidx, posbuf, acc,
              gsem, ssem):
    c = lax.axis_index("c")
    s = lax.axis_index("s")
    pltpu.sync_copy(zeros_hbm, acc.at[pl.ds(s * TROWS, TROWS)])

    base = c * HALFR
    pos0 = jnp.zeros((16,), jnp.int32)

    def phase(ph, pos):
        pltpu.sync_copy(edges_hbm.at[s, :, pl.ds(ph * PH, PH)], idx_stage)

        def prow(k, pos):
            for j in range(CHUNK // 16):
                srcv = idx_stage[0, k, pl.ds(j * 16, 16)]
                dstv = idx_stage[1, k, pl.ds(j * 16, 16)]
                dl = dstv - base
                mask = (dl >= 0) & (dl < HALFR)
                prefix = plsc.cumsum(mask.astype(jnp.int32))
                positions = pos + prefix - 1
                plsc.store_scatter(kept_src, [positions], srcv, mask=mask)
                plsc.store_scatter(kept_dst, [positions], dl, mask=mask)
                pos = pos + plsc.all_reduce_population_count(mask)
            return pos

        return lax.fori_loop(0, PH, prow, pos)

    pos = lax.fori_loop(0, SSTEPS // PH, phase, pos0)

    iot = lax.iota(jnp.int32, 16)
    padsrc = jnp.full((16,), N, jnp.int32)
    paddst = jnp.full((16,), DUMMY, jnp.int32)
    for j in range(NBUF * CHUNK // 16):
        plsc.store_scatter(kept_src, [pos + iot + j * 16], padsrc)
        plsc.store_scatter(kept_dst, [pos + iot + j * 16], paddst)
    posbuf[...] = pos
    cnt = posbuf[...][0]
    nblk = (cnt + NBUF * CHUNK - 1) // (NBUF * CHUNK)

    plsc.subcore_barrier()

    def block(t, carry):
        gds = []
        for j in range(NBUF):
            off = (t * NBUF + j) * CHUNK
            gds.append(pltpu.async_copy(
                table_hbm.at[kept_src.at[pl.ds(off, CHUNK)]],
                rows.at[j], gsem.at[j]))
        sds = []
        for j in range(NBUF):
            off = (t * NBUF + j) * CHUNK
            for i in range(CHUNK // 16):
                sidx[j, pl.ds(i * 16, 16)] = kept_dst[pl.ds(off + i * 16, 16)]
            gds[j].wait()
            sds.append(pltpu.async_copy(
                rows.at[j], acc.at[sidx.at[j]], ssem.at[j], add=True))
        for d in sds:
            d.wait()
        return carry

    lax.fori_loop(0, nblk, block, 0)
    plsc.subcore_barrier()
    pltpu.sync_copy(acc.at[pl.ds(s * TROWS, TROWS)],
                    out_hbm.at[c, pl.ds(s * TROWS, TROWS)])


@functools.lru_cache(maxsize=None)
def _agg_kernel():
    return pl.kernel(
        _agg_body,
        out_type=jax.ShapeDtypeStruct((NC, ACCR, D), jnp.float32),
        mesh=_mesh(),
        compiler_params=pltpu.CompilerParams(needs_layout_passes=False),
        scratch_types=[
            pltpu.VMEM((2, PH, CHUNK), jnp.int32),
            pltpu.VMEM((CAP,), jnp.int32),
            pltpu.VMEM((CAP,), jnp.int32),
            pltpu.VMEM((NBUF, CHUNK, D), jnp.float32),
            pltpu.VMEM((NBUF, CHUNK), jnp.int32),
            pltpu.VMEM((16,), jnp.int32),
            pltpu.VMEM_SHARED((ACCR, D), jnp.float32),
            pltpu.SemaphoreType.DMA((NBUF,)),
            pltpu.SemaphoreType.DMA((NBUF,)),
        ],
    )


def _first_body(x_ref, w_ref, dp_ref, hpre_ref, hs_ref, dinv_ref):
    dsum = lax.dot_general(dp_ref[...], jnp.ones((NW, 1), jnp.float32),
                           (((0,), (0,)), ((), ())),
                           preferred_element_type=jnp.float32)
    deg = dsum[0:N, :] + 1.0
    dinv = lax.rsqrt(deg)
    dinv_ref[...] = dinv
    h = jnp.dot(x_ref[...], w_ref[...], preferred_element_type=jnp.float32)
    hpre_ref[...] = h
    hs_ref[0:N, :] = h * dinv
    hs_ref[N:NP, :] = jnp.zeros((NP - N, D), jnp.float32)


_first_kernel = pl.pallas_call(
    _first_body,
    out_shape=(
        jax.ShapeDtypeStruct((N, D), jnp.float32),
        jax.ShapeDtypeStruct((NP, D), jnp.float32),
        jax.ShapeDtypeStruct((N, 1), jnp.float32),
    ),
)


def _mid_body(p0_ref, p1_ref, hpre_ref, dinv_ref, b_ref, g_ref, be_ref,
              rm_ref, rv_ref, w_ref, hpre_o_ref, hs_o_ref):
    dinv = dinv_ref[...]
    psum = jnp.concatenate([p0_ref[0:HALFR, :], p1_ref[0:N - HALFR, :]],
                           axis=0)
    agg = psum * dinv + hpre_ref[...] * (dinv * dinv) + b_ref[...]
    o = (agg - rm_ref[...]) * lax.rsqrt(rv_ref[...] + 1e-5) * g_ref[...] \
        + be_ref[...]
    o = jnp.maximum(o, 0.0)
    h = jnp.dot(o, w_ref[...], preferred_element_type=jnp.float32)
    hpre_o_ref[...] = h
    hs_o_ref[0:N, :] = h * dinv
    hs_o_ref[N:NP, :] = jnp.zeros((NP - N, D), jnp.float32)


_mid_kernel = pl.pallas_call(
    _mid_body,
    out_shape=(
        jax.ShapeDtypeStruct((N, D), jnp.float32),
        jax.ShapeDtypeStruct((NP, D), jnp.float32),
    ),
)


def _final_body(p0_ref, p1_ref, hpre_ref, dinv_ref, b_ref, batch_ref,
                wout_ref, bout_ref, out_ref):
    dinv = dinv_ref[...]
    psum = jnp.concatenate([p0_ref[0:HALFR, :], p1_ref[0:N - HALFR, :]],
                           axis=0)
    o = psum * dinv + hpre_ref[...] * (dinv * dinv) + b_ref[...]
    gids = lax.broadcasted_iota(jnp.int32, (1, G), 1)
    onehot = jnp.where(batch_ref[...] == gids, 1.0, 0.0)
    sums = lax.dot_general(onehot, o, (((0,), (0,)), ((), ())),
                           preferred_element_type=jnp.float32)
    counts = lax.dot_general(onehot, jnp.ones((N, 1), jnp.float32),
                             (((0,), (0,)), ((), ())),
                             preferred_element_type=jnp.float32)
    pooled = sums / jnp.maximum(counts, 1.0)
    logits = jnp.dot(pooled, wout_ref[...],
                     preferred_element_type=jnp.float32) + bout_ref[...]
    m = jnp.max(logits, axis=1, keepdims=True)
    z = logits - m
    lse = jnp.log(jnp.sum(jnp.exp(z), axis=1, keepdims=True))
    out_ref[...] = z - lse


_final_kernel = pl.pallas_call(
    _final_body,
    out_shape=jax.ShapeDtypeStruct((G, D_OUT), jnp.float32),
)


def kernel(x, edge_index, batch, Ws, bs, gammas, betas, rms, rvs, Wout, bout):
    pad = jnp.full((EP - E,), N, jnp.int32)
    src_p = jnp.concatenate([edge_index[0].astype(jnp.int32), pad])
    dst_p = jnp.concatenate([edge_index[1].astype(jnp.int32), pad])
    dst_r = dst_p.reshape(NW, STEPS, CHUNK)

    pad_agg = jnp.full((EP - E,), PADDST, jnp.int32)
    dst_a = jnp.concatenate([edge_index[1].astype(jnp.int32), pad_agg])
    edges_ns = jnp.stack([src_p.reshape(NS, SSTEPS, CHUNK),
                          dst_a.reshape(NS, SSTEPS, CHUNK)], axis=1)

    zerosD = jnp.zeros((TROWS, D), jnp.float32)

    degp = _deg_kernel()(dst_r)

    hpre, hs, dinv = _first_kernel(x.astype(jnp.float32), Ws[0], degp)

    for i in range(1, NUM_LAYERS):
        p = _agg_kernel()(edges_ns, hs, zerosD)
        j = i - 1
        hpre, hs = _mid_kernel(
            p[0], p[1], hpre, dinv,
            bs[j].reshape(1, D), gammas[j].reshape(1, D),
            betas[j].reshape(1, D), rms[j].reshape(1, D),
            rvs[j].reshape(1, D), Ws[i])

    p = _agg_kernel()(edges_ns, hs, zerosD)
    return _final_kernel(p[0], p[1], hpre, dinv,
                         bs[NUM_LAYERS - 1].reshape(1, D),
                         batch.reshape(N, 1).astype(jnp.int32),
                         Wout, bout.reshape(1, D_OUT))
